# baseline (speedup 1.0000x reference)
import jax
import jax.numpy as jnp
from jax import lax
from jax.experimental import pallas as pl
from jax.experimental.pallas import tpu as pltpu

B = 1024
N_PATCHES = 128
D_MODEL = 768
BB = 32
NCH = B // BB
RG = 4
NSLOT = 2


def _body(rp_ref, w_ref, s_ref, out_hbm, buf, sems):
    wv = (w_ref[...] * s_ref[0]).reshape(1, 1, D_MODEL)

    def chunk_pair(it, carry):
        for sl in range(NSLOT):
            ci = it * NSLOT + sl

            @pl.when(ci >= NSLOT)
            def _reclaim():
                pltpu.make_async_copy(
                    buf.at[sl], out_hbm.at[pl.ds(ci * BB, BB), :, :], sems.at[sl]
                ).wait()

            def row_body(j, c2):
                buf[sl, pl.ds(j * RG, RG)] = (
                    rp_ref[pl.ds(ci * BB + j * RG, RG), :][:, :, None] * wv
                )
                return c2

            lax.fori_loop(0, BB // RG, row_body, 0)

            pltpu.make_async_copy(
                buf.at[sl], out_hbm.at[pl.ds(ci * BB, BB), :, :], sems.at[sl]
            ).start()
        return carry

    lax.fori_loop(0, NCH // NSLOT, chunk_pair, 0)

    for sl in range(NSLOT):
        pltpu.make_async_copy(
            buf.at[sl], out_hbm.at[pl.ds(0, BB), :, :], sems.at[sl]
        ).wait()


def kernel(n_patches, relative_positions, W, scale):
    w2 = W.reshape(1, D_MODEL)
    out = pl.pallas_call(
        _body,
        grid=(1,),
        in_specs=[
            pl.BlockSpec((B, N_PATCHES), lambda i: (0, 0)),
            pl.BlockSpec((1, D_MODEL), lambda i: (0, 0)),
            pl.BlockSpec(memory_space=pltpu.SMEM),
        ],
        out_specs=pl.BlockSpec(memory_space=pl.ANY),
        out_shape=jax.ShapeDtypeStruct((B, N_PATCHES, D_MODEL), jnp.float32),
        scratch_shapes=[
            pltpu.VMEM((NSLOT, BB, N_PATCHES, D_MODEL), jnp.float32),
            pltpu.SemaphoreType.DMA((NSLOT,)),
        ],
    )(relative_positions, w2, scale)
    return out


# speedup vs baseline: 1.0090x; 1.0090x over previous
import jax
import jax.numpy as jnp
from jax.experimental import pallas as pl
from jax.experimental.pallas import tpu as pltpu

B = 1024
N_PATCHES = 128
D_MODEL = 768
BB = 32


def _body(rp_ref, w_ref, s_ref, out_ref):
    wv = (w_ref[...] * s_ref[0]).reshape(1, 1, D_MODEL)
    out_ref[...] = rp_ref[...][:, :, None] * wv


def kernel(n_patches, relative_positions, W, scale):
    w2 = W.reshape(1, D_MODEL)
    grid = (B // BB,)
    out = pl.pallas_call(
        _body,
        grid=grid,
        in_specs=[
            pl.BlockSpec((BB, N_PATCHES), lambda i: (i, 0)),
            pl.BlockSpec((1, D_MODEL), lambda i: (0, 0)),
            pl.BlockSpec(memory_space=pltpu.SMEM),
        ],
        out_specs=pl.BlockSpec((BB, N_PATCHES, D_MODEL), lambda i: (i, 0, 0)),
        out_shape=jax.ShapeDtypeStruct((B, N_PATCHES, D_MODEL), jnp.float32),
        compiler_params=pltpu.CompilerParams(
            dimension_semantics=("parallel",),
        ),
    )(relative_positions, w2, scale)
    return out
